# baseline (speedup 1.0000x reference)
import jax
import jax.numpy as jnp
from jax.experimental import pallas as pl
from jax.experimental.pallas import tpu as pltpu


def _read_body(v_ref, t_ref, s_ref):
    acc = jnp.sum(v_ref[0]) + jnp.sum(t_ref[0])
    s_ref[0] = jnp.zeros((8, 128), jnp.float32) + acc


def kernel(visual_feat, tactile_feat, Wp, bp, edge_index):
    B, Cv, H, W = visual_feat.shape
    Ct = tactile_feat.shape[1]
    CH = 128

    s_out = pl.pallas_call(
        _read_body,
        grid=(B, Cv // CH),
        in_specs=[
            pl.BlockSpec((1, CH, H, W), lambda b, k: (b, k, 0, 0)),
            pl.BlockSpec((1, CH, H, W), lambda b, k: (b, k, 0, 0)),
        ],
        out_specs=pl.BlockSpec((1, 8, 128), lambda b, k: (b, 0, 0)),
        out_shape=jax.ShapeDtypeStruct((B, 8, 128), jnp.float32),
    )(visual_feat, tactile_feat)

    return (s_out, s_out, s_out)


# speedup vs baseline: 6.6107x; 6.6107x over previous
import jax
import jax.numpy as jnp
from jax.experimental import pallas as pl
from jax.experimental.pallas import tpu as pltpu


def _tc_body(vis_ref, tac_ref, wv_ref, wt_ref, bp_ref, ei_ref,
             x_ref, attr_ref, eib_ref, mt_ref):
    b = pl.program_id(0)
    N, E = mt_ref.shape
    cv = vis_ref.shape[2]

    @pl.when(b == 0)
    def _build_incidence():
        ids = jax.lax.broadcasted_iota(jnp.int32, (N, E), 0)
        s = ei_ref[0:1, :]
        d = ei_ref[1:2, :]
        mt_ref[...] = (ids == s).astype(jnp.float32) - (ids == d).astype(jnp.float32)

    v = vis_ref[0]
    t = tac_ref[0]
    x_ref[0, :, 0:cv] = v
    x_ref[0, :, cv:] = t
    dims = (((0,), (1,)), ((), ()))
    cT = (jax.lax.dot_general(wv_ref[...], v, dims,
                              preferred_element_type=jnp.float32)
          + jax.lax.dot_general(wt_ref[...], t, dims,
                                preferred_element_type=jnp.float32)
          + bp_ref[...])
    diffT = jnp.dot(cT, mt_ref[...], preferred_element_type=jnp.float32)
    dx = diffT[0:1, :]
    dy = diffT[1:2, :]
    dist = jnp.sqrt(dx * dx + dy * dy)
    w = 1.0 / (dist + 1e-6)
    attr_ref[0] = 1.0 / (1.0 + jnp.exp(-w))
    eib_ref[0] = ei_ref[...] + (b * N).astype(ei_ref.dtype)


def kernel(visual_feat, tactile_feat, Wp, bp, edge_index):
    B, Cv, H, W = visual_feat.shape
    Ct = tactile_feat.shape[1]
    C = Cv + Ct
    N = H * W
    E = edge_index.shape[1]

    vis = jnp.transpose(visual_feat, (0, 2, 3, 1)).reshape(B, N, Cv)
    tac = jnp.transpose(tactile_feat, (0, 2, 3, 1)).reshape(B, N, Ct)
    wv = Wp[:Cv]
    wt = Wp[Cv:]
    bp2 = bp.reshape(2, 1)
    ei = edge_index.astype(jnp.int32)

    in_specs = [
        pl.BlockSpec((1, N, Cv), lambda b: (b, 0, 0)),
        pl.BlockSpec((1, N, Ct), lambda b: (b, 0, 0)),
        pl.BlockSpec((Cv, 2), lambda b: (0, 0)),
        pl.BlockSpec((Ct, 2), lambda b: (0, 0)),
        pl.BlockSpec((2, 1), lambda b: (0, 0)),
        pl.BlockSpec((2, E), lambda b: (0, 0)),
    ]
    out_specs = [
        pl.BlockSpec((1, N, C), lambda b: (b, 0, 0)),
        pl.BlockSpec((1, 1, E), lambda b: (b, 0, 0)),
        pl.BlockSpec((1, 2, E), lambda b: (b, 0, 0)),
    ]

    x_out, attr_out, eib_out = pl.pallas_call(
        _tc_body,
        grid=(B,),
        in_specs=in_specs,
        out_specs=out_specs,
        out_shape=[
            jax.ShapeDtypeStruct((B, N, C), jnp.float32),
            jax.ShapeDtypeStruct((B, 1, E), jnp.float32),
            jax.ShapeDtypeStruct((B, 2, E), edge_index.dtype),
        ],
        scratch_shapes=[pltpu.VMEM((N, E), jnp.float32)],
    )(vis, tac, wv, wt, bp2, ei)

    x_batched = x_out.reshape(B * N, C)
    edge_index_batched = eib_out.transpose(1, 0, 2).reshape(2, B * E)
    edge_attr_batched = attr_out.reshape(B * E, 1)
    return (x_batched, edge_index_batched, edge_attr_batched)
